# baseline (speedup 1.0000x reference)
import jax
import jax.numpy as jnp
from jax import lax
from jax.experimental import pallas as pl
from jax.experimental.pallas import tpu as pltpu
from jax.experimental.pallas import tpu_sc as plsc

D_MODEL = 1024
GATE_DIM = D_MODEL // 2
POS_DIM = D_MODEL // 4
PARAM_DIM = D_MODEL - GATE_DIM - POS_DIM
SEQ_LEN = 8192
N_GATE_TYPES = 20

NC = 2
NS = 16
L = 16
NW = NC * NS
ROWS_PER_W = SEQ_LEN // NW
CHUNK = 32
NCHUNKS = ROWS_PER_W // CHUNK
NBUF = 3


def _body(x_h, gid_h, gate_h, w_h, bp_h, bq_h, pos_h, out_h,
          b0, b1, b2, gatebuf, wbuf, biasbuf, xbuf, gidb,
          si0, si1, si2, so0, so1, so2):
    wid = lax.axis_index("s") * NC + lax.axis_index("c")
    base = wid * ROWS_PER_W
    bufs = (b0, b1, b2)
    sin = (si0, si1, si2)
    sout = (so0, so1, so2)

    pltpu.sync_copy(gate_h, gatebuf)
    pltpu.sync_copy(w_h, wbuf)
    pltpu.sync_copy(bp_h, biasbuf.at[pl.ds(0, POS_DIM)])
    pltpu.sync_copy(bq_h, biasbuf.at[pl.ds(POS_DIM, PARAM_DIM)])
    pltpu.sync_copy(x_h.at[pl.ds(base * 4, ROWS_PER_W * 4)], xbuf)
    pltpu.sync_copy(gid_h.at[pl.ds(base, ROWS_PER_W)],
                    gidb.at[pl.ds(0, ROWS_PER_W)])

    vc = [jnp.full((L,), c, jnp.int32) for c in range(4)]

    def in_slice(c):
        return pos_h.at[pl.ds(base + c * CHUNK, CHUNK), :]

    def out_slice(c):
        return out_h.at[pl.ds(base + c * CHUNK, CHUNK), :]

    def compute(buf, c):
        @plsc.parallel_loop(0, CHUNK)
        def row(r):
            ri = c * CHUNK + r
            vr = jnp.full((L,), ri * 4, jnp.int32)
            v1 = plsc.load_gather(xbuf, [vr + vc[1]])
            v2 = plsc.load_gather(xbuf, [vr + vc[2]])
            v3 = plsc.load_gather(xbuf, [vr + vc[3]])
            g = gidb[pl.ds(ri, L)][0]

            @plsc.parallel_loop(0, GATE_DIM // L, unroll=8)
            def gate_k(k):
                sl = pl.ds(k * L, L)
                plsc.addupdate(buf.at[r, sl], gatebuf[g, sl])

            @plsc.parallel_loop(0, POS_DIM // L, unroll=8)
            def pos_k(k):
                sl = pl.ds(GATE_DIM + k * L, L)
                t = biasbuf[pl.ds(k * L, L)] + (
                    v1 * wbuf[pl.ds(k * L, L)]
                    + v2 * wbuf[pl.ds(POS_DIM + k * L, L)])
                plsc.addupdate(buf.at[r, sl], t)

            @plsc.parallel_loop(0, PARAM_DIM // L, unroll=8)
            def param_k(k):
                sl = pl.ds(GATE_DIM + POS_DIM + k * L, L)
                t = biasbuf[pl.ds(POS_DIM + k * L, L)] + (
                    v3 * wbuf[pl.ds(2 * POS_DIM + k * L, L)])
                plsc.addupdate(buf.at[r, sl], t)

    for j in range(NBUF - 1):
        pltpu.async_copy(in_slice(j), bufs[j], sin[j])
    for c in range(NCHUNKS):
        j = c % NBUF
        jn = (c + NBUF - 1) % NBUF
        pltpu.make_async_copy(in_slice(c), bufs[j], sin[j]).wait()
        compute(bufs[j], c)
        pltpu.async_copy(bufs[j], out_slice(c), sout[j])
        if 1 <= c <= NCHUNKS - NBUF:
            pltpu.make_async_copy(bufs[jn], out_slice(c - 1), sout[jn]).wait()
        if c <= NCHUNKS - NBUF:
            pltpu.async_copy(in_slice(c + NBUF - 1), bufs[jn], sin[jn])
    for c in range(NCHUNKS - NBUF, NCHUNKS):
        pltpu.make_async_copy(bufs[c % NBUF], out_slice(c),
                              sout[c % NBUF]).wait()


_sc_call = pl.kernel(
    _body,
    out_type=jax.ShapeDtypeStruct((SEQ_LEN, D_MODEL), jnp.float32),
    mesh=plsc.VectorSubcoreMesh(core_axis_name="c", subcore_axis_name="s",
                                num_cores=NC, num_subcores=NS),
    compiler_params=pltpu.CompilerParams(needs_layout_passes=False),
    scratch_types=(
        [pltpu.VMEM((CHUNK, D_MODEL), jnp.float32) for _ in range(NBUF)]
        + [pltpu.VMEM((N_GATE_TYPES, GATE_DIM), jnp.float32),
           pltpu.VMEM((2 * POS_DIM + PARAM_DIM,), jnp.float32),
           pltpu.VMEM((POS_DIM + PARAM_DIM,), jnp.float32),
           pltpu.VMEM((ROWS_PER_W * 4,), jnp.float32),
           pltpu.VMEM((ROWS_PER_W + L,), jnp.int32)]
        + [pltpu.SemaphoreType.DMA for _ in range(2 * NBUF)]
    ),
)


def kernel(x, gate_table, pos_table, W_pos, b_pos, W_param, b_param):
    wcat = jnp.concatenate([W_pos[:, 0], W_pos[:, 1], W_param[:, 0]])
    gid = x[:, 0].astype(jnp.int32)
    return _sc_call(x.reshape(-1), gid, gate_table, wcat, b_pos, b_param,
                    pos_table)
